# baseline (speedup 1.0000x reference)
import functools

import jax
import jax.numpy as jnp
from jax import lax
from jax.experimental import pallas as pl
from jax.experimental.pallas import tpu as pltpu
from jax.experimental.pallas import tpu_sc as plsc

N = 10000
NP = 10240
E = 320000
D = 128
NC = 2
NS = 16
NW = NC * NS
EPT = E // NW
B = 128
JT = NP // B
GRP = 16
ROWS_PT = NP // NS

_MESH = plsc.VectorSubcoreMesh(core_axis_name="c", subcore_axis_name="s")


def _sc_degree(dstb, ones_b, zeros_rows):
  @functools.partial(
      pl.kernel,
      out_type=jax.ShapeDtypeStruct((NC, NP, D), jnp.float32),
      mesh=_MESH,
      scratch_types=[
          pltpu.VMEM((JT, B), jnp.int32),
          pltpu.VMEM((B, D), jnp.float32),
          pltpu.VMEM_SHARED((NP, D), jnp.float32),
      ],
  )
  def k(dst_hbm, ones_hbm, zeros_hbm, out_hbm, didx, ones_v, acc):
    cid = lax.axis_index("c")
    sid = lax.axis_index("s")
    wid = cid * NS + sid
    pltpu.sync_copy(zeros_hbm, acc.at[pl.ds(sid * ROWS_PT, ROWS_PT)])
    pltpu.sync_copy(ones_hbm, ones_v)
    pltpu.sync_copy(dst_hbm.at[wid], didx)
    plsc.subcore_barrier()

    @pl.loop(0, JT)
    def _(j):
      pltpu.sync_copy(ones_v, acc.at[didx.at[j]], add=True)

    plsc.subcore_barrier()
    pltpu.sync_copy(
        acc.at[pl.ds(sid * ROWS_PT, ROWS_PT)],
        out_hbm.at[cid, pl.ds(sid * ROWS_PT, ROWS_PT)],
    )

  return k(dstb, ones_b, zeros_rows)


def _sc_aggregate(hp, srcb, dstb, zeros_rows):
  @functools.partial(
      pl.kernel,
      out_type=jax.ShapeDtypeStruct((NC, NP, D), jnp.float32),
      mesh=_MESH,
      scratch_types=[
          pltpu.VMEM((B,), jnp.int32),
          pltpu.VMEM((B,), jnp.int32),
          pltpu.VMEM((B, D), jnp.float32),
          pltpu.VMEM_SHARED((NP, D), jnp.float32),
      ],
  )
  def k(h_hbm, src_hbm, dst_hbm, zeros_hbm, out_hbm, sidx, didx, rows, acc):
    cid = lax.axis_index("c")
    sid = lax.axis_index("s")
    wid = cid * NS + sid
    pltpu.sync_copy(zeros_hbm, acc.at[pl.ds(sid * ROWS_PT, ROWS_PT)])
    plsc.subcore_barrier()

    @pl.loop(0, JT)
    def _(j):
      pltpu.sync_copy(src_hbm.at[wid, j], sidx)
      pltpu.sync_copy(dst_hbm.at[wid, j], didx)
      pltpu.sync_copy(h_hbm.at[sidx], rows)
      pltpu.sync_copy(rows, acc.at[didx], add=True)

    plsc.subcore_barrier()
    pltpu.sync_copy(
        acc.at[pl.ds(sid * ROWS_PT, ROWS_PT)],
        out_hbm.at[cid, pl.ds(sid * ROWS_PT, ROWS_PT)],
    )

  return k(hp, srcb, dstb, zeros_rows)


def _tc_matmul(x, w):
  def body(x_ref, w_ref, o_ref):
    o_ref[...] = jnp.dot(x_ref[...], w_ref[...],
                         preferred_element_type=jnp.float32)

  return pl.pallas_call(
      body, out_shape=jax.ShapeDtypeStruct((NP, D), jnp.float32))(x, w)


def _dinv(dp_ref):
  deg = dp_ref[0, :, 0:1] + dp_ref[1, :, 0:1] + 1.0
  return lax.rsqrt(deg)


def _tc_scale(m, degparts):
  def body(m_ref, dp_ref, o_ref):
    o_ref[...] = m_ref[...] * _dinv(dp_ref)

  return pl.pallas_call(
      body, out_shape=jax.ShapeDtypeStruct((NP, D), jnp.float32))(m, degparts)


def _tc_mid(parts1, h1p, degparts, w2, b1):
  def body(p_ref, h_ref, dp_ref, w_ref, b_ref, o_ref):
    dinv = _dinv(dp_ref)
    z = (p_ref[0] + p_ref[1] + h_ref[...]) * dinv + b_ref[...]
    z = jnp.maximum(z, 0.0)
    m2 = jnp.dot(z, w_ref[...], preferred_element_type=jnp.float32)
    o_ref[...] = m2 * dinv

  return pl.pallas_call(
      body, out_shape=jax.ShapeDtypeStruct((NP, D), jnp.float32))(
          parts1, h1p, degparts, w2, b1)


def _tc_final(parts2, h2p, degparts, b2):
  def body(p_ref, h_ref, dp_ref, b_ref, o_ref):
    o_ref[...] = (p_ref[0] + p_ref[1] + h_ref[...]) * _dinv(dp_ref) + b_ref[...]

  return pl.pallas_call(
      body, out_shape=jax.ShapeDtypeStruct((NP, D), jnp.float32))(
          parts2, h2p, degparts, b2)


def _block_edges(e):
  e = e.reshape(NW, EPT)
  pad = jnp.full((NW, JT * B - EPT), N, jnp.int32)
  return jnp.concatenate([e, pad], axis=1).reshape(NW, JT, B)


@jax.jit
def kernel(x, edge_index, W1, b1, W2, b2):
  srcb = _block_edges(edge_index[0].astype(jnp.int32))
  dstb = _block_edges(edge_index[1].astype(jnp.int32))
  xp = jnp.pad(x, ((0, NP - N), (0, 0)))
  ones_b = jnp.ones((B, D), jnp.float32)
  zeros_rows = jnp.zeros((ROWS_PT, D), jnp.float32)

  degparts = _sc_degree(dstb, ones_b, zeros_rows)
  m1 = _tc_matmul(xp, W1)
  h1p = _tc_scale(m1, degparts)
  parts1 = _sc_aggregate(h1p, srcb, dstb, zeros_rows)
  h2p = _tc_mid(parts1, h1p, degparts, W2, b1.reshape(1, D))
  parts2 = _sc_aggregate(h2p, srcb, dstb, zeros_rows)
  out = _tc_final(parts2, h2p, degparts, b2.reshape(1, D))
  return out[:N]


# speedup vs baseline: 2.1113x; 2.1113x over previous
import functools

import jax
import jax.numpy as jnp
from jax import lax
from jax.experimental import pallas as pl
from jax.experimental.pallas import tpu as pltpu
from jax.experimental.pallas import tpu_sc as plsc

N = 10000
NP = 10240
E = 320000
D = 128
NC = 2
NS = 16
NW = NC * NS
EPT = E // NW
B = 128
JT = NP // B
EB = 80
ROWS_PT = NP // NS

_MESH = plsc.VectorSubcoreMesh(core_axis_name="c", subcore_axis_name="s")


def _sc_degree(dstb, ones_b, zeros_rows):
  @functools.partial(
      pl.kernel,
      out_type=jax.ShapeDtypeStruct((NC, NP, D), jnp.float32),
      mesh=_MESH,
      scratch_types=[
          pltpu.VMEM((JT, B), jnp.int32),
          pltpu.VMEM((B, D), jnp.float32),
          pltpu.VMEM_SHARED((NP, D), jnp.float32),
      ],
  )
  def k(dst_hbm, ones_hbm, zeros_hbm, out_hbm, didx, ones_v, acc):
    cid = lax.axis_index("c")
    sid = lax.axis_index("s")
    wid = cid * NS + sid
    pltpu.sync_copy(zeros_hbm, acc.at[pl.ds(sid * ROWS_PT, ROWS_PT)])
    pltpu.sync_copy(ones_hbm, ones_v)
    pltpu.sync_copy(dst_hbm.at[wid], didx)
    plsc.subcore_barrier()

    @pl.loop(0, JT)
    def _(j):
      pltpu.sync_copy(ones_v, acc.at[didx.at[j]], add=True)

    plsc.subcore_barrier()
    pltpu.sync_copy(
        acc.at[pl.ds(sid * ROWS_PT, ROWS_PT)],
        out_hbm.at[cid, pl.ds(sid * ROWS_PT, ROWS_PT)],
    )

  return k(dstb, ones_b, zeros_rows)


def _sc_aggregate(hp, src, dst, zeros_rows):
  @functools.partial(
      pl.kernel,
      out_type=jax.ShapeDtypeStruct((NC, NP, D), jnp.float32),
      mesh=_MESH,
      scratch_types=[
          pltpu.VMEM((EPT,), jnp.int32),
          pltpu.VMEM((EPT,), jnp.int32),
          pltpu.VMEM((EB, D), jnp.float32),
          pltpu.VMEM_SHARED((NP, D), jnp.float32),
      ],
  )
  def k(h_hbm, src_hbm, dst_hbm, zeros_hbm, out_hbm, sidx, didx, rows, acc):
    cid = lax.axis_index("c")
    sid = lax.axis_index("s")
    wid = cid * NS + sid
    pltpu.sync_copy(zeros_hbm, acc.at[pl.ds(sid * ROWS_PT, ROWS_PT)])
    pltpu.sync_copy(src_hbm.at[pl.ds(wid * EPT, EPT)], sidx)
    pltpu.sync_copy(dst_hbm.at[pl.ds(wid * EPT, EPT)], didx)
    plsc.subcore_barrier()

    @pl.loop(0, EPT, step=EB)
    def _(e):
      pltpu.sync_copy(h_hbm.at[sidx.at[pl.ds(e, EB)]], rows)
      pltpu.sync_copy(rows, acc.at[didx.at[pl.ds(e, EB)]], add=True)

    plsc.subcore_barrier()
    pltpu.sync_copy(
        acc.at[pl.ds(sid * ROWS_PT, ROWS_PT)],
        out_hbm.at[cid, pl.ds(sid * ROWS_PT, ROWS_PT)],
    )

  return k(hp, src, dst, zeros_rows)


def _tc_matmul(x, w):
  def body(x_ref, w_ref, o_ref):
    o_ref[...] = jnp.dot(x_ref[...], w_ref[...],
                         preferred_element_type=jnp.float32)

  return pl.pallas_call(
      body, out_shape=jax.ShapeDtypeStruct((NP, D), jnp.float32))(x, w)


def _dinv(dp_ref):
  deg = dp_ref[0, :, 0:1] + dp_ref[1, :, 0:1] + 1.0
  return lax.rsqrt(deg)


def _tc_scale(m, degparts):
  def body(m_ref, dp_ref, o_ref):
    o_ref[...] = m_ref[...] * _dinv(dp_ref)

  return pl.pallas_call(
      body, out_shape=jax.ShapeDtypeStruct((NP, D), jnp.float32))(m, degparts)


def _tc_mid(parts1, h1p, degparts, w2, b1):
  def body(p_ref, h_ref, dp_ref, w_ref, b_ref, o_ref):
    dinv = _dinv(dp_ref)
    z = (p_ref[0] + p_ref[1] + h_ref[...]) * dinv + b_ref[...]
    z = jnp.maximum(z, 0.0)
    m2 = jnp.dot(z, w_ref[...], preferred_element_type=jnp.float32)
    o_ref[...] = m2 * dinv

  return pl.pallas_call(
      body, out_shape=jax.ShapeDtypeStruct((NP, D), jnp.float32))(
          parts1, h1p, degparts, w2, b1)


def _tc_final(parts2, h2p, degparts, b2):
  def body(p_ref, h_ref, dp_ref, b_ref, o_ref):
    o_ref[...] = (p_ref[0] + p_ref[1] + h_ref[...]) * _dinv(dp_ref) + b_ref[...]

  return pl.pallas_call(
      body, out_shape=jax.ShapeDtypeStruct((NP, D), jnp.float32))(
          parts2, h2p, degparts, b2)


def _block_edges(e):
  e = e.reshape(NW, EPT)
  pad = jnp.full((NW, JT * B - EPT), N, jnp.int32)
  return jnp.concatenate([e, pad], axis=1).reshape(NW, JT, B)


@jax.jit
def kernel(x, edge_index, W1, b1, W2, b2):
  src = edge_index[0].astype(jnp.int32)
  dst = edge_index[1].astype(jnp.int32)
  dstb = _block_edges(dst)
  xp = jnp.pad(x, ((0, NP - N), (0, 0)))
  ones_b = jnp.ones((B, D), jnp.float32)
  zeros_rows = jnp.zeros((ROWS_PT, D), jnp.float32)

  degparts = _sc_degree(dstb, ones_b, zeros_rows)
  m1 = _tc_matmul(xp, W1)
  h1p = _tc_scale(m1, degparts)
  parts1 = _sc_aggregate(h1p, src, dst, zeros_rows)
  h2p = _tc_mid(parts1, h1p, degparts, W2, b1.reshape(1, D))
  parts2 = _sc_aggregate(h2p, src, dst, zeros_rows)
  out = _tc_final(parts2, h2p, degparts, b2.reshape(1, D))
  return out[:N]
